# baseline (speedup 1.0000x reference)
import jax
import jax.numpy as jnp
from jax import lax
from jax.experimental import pallas as pl
from jax.experimental.pallas import tpu as pltpu
from jax.experimental.pallas import tpu_sc as plsc

N = 100_000
E = 3_200_000
F = 16
NC = 2
NS = 16
NW = NC * NS
CH = 128
CROWS = 25_600
EPAD = CROWS * CH
NCHUNK = CROWS // NW
NBUF = 4
SBD = 100
NSTD = NCHUNK // SBD
NGD = SBD // NBUF
SBM = 40
NSTM = NCHUNK // SBM
NGM = SBM // NBUF

RPT = 6_272
NPAD = NS * RPT
ZR = 128
NZC = RPT // ZR

DPT = 6_272
NPD = NS * DPT
NR = NPD // 128

_MESH = plsc.VectorSubcoreMesh(core_axis_name="c", subcore_axis_name="s")


def _sc_deg_body(dst_hbm, out_hbm, dstb, onesb, zb, acc, s0, s1, s2, s3):
    ssc = (s0, s1, s2, s3)
    cid = lax.axis_index("c")
    sid = lax.axis_index("s")
    zeros16 = jnp.zeros((16,), jnp.float32)
    ones16 = jnp.ones((16,), jnp.float32)
    for i in range(CH // 16):
        onesb[pl.ds(i * 16, 16)] = ones16

    def zfill(i, _):
        zb[pl.ds(i * 16, 16)] = zeros16
        return 0

    lax.fori_loop(0, DPT // 16, zfill, 0)
    pltpu.sync_copy(zb, acc.at[pl.ds(sid * DPT, DPT)])
    plsc.subcore_barrier()

    crow0 = (cid * NS + sid) * NCHUNK
    for h in range(NSTD):
        pltpu.sync_copy(dst_hbm.at[pl.ds(crow0 + h * SBD, SBD)], dstb)
        for b in range(NBUF):
            pltpu.async_copy(onesb, acc.at[dstb.at[b]], ssc[b], add=True)

        def grp(t, _):
            for b in range(NBUF):
                u = t * NBUF + b
                pltpu.make_async_copy(onesb, acc.at[dstb.at[u]],
                                      ssc[b]).wait()
                pltpu.async_copy(onesb, acc.at[dstb.at[u + NBUF]],
                                 ssc[b], add=True)
            return 0

        lax.fori_loop(0, NGD - 1, grp, 0)
        for b in range(NBUF):
            u = (NGD - 1) * NBUF + b
            pltpu.make_async_copy(onesb, acc.at[dstb.at[u]], ssc[b]).wait()
    plsc.subcore_barrier()
    pltpu.sync_copy(acc.at[pl.ds(sid * DPT, DPT)],
                    out_hbm.at[pl.ds(cid * NPD + sid * DPT, DPT)])


_sc_deg = pl.kernel(
    _sc_deg_body,
    out_type=jax.ShapeDtypeStruct((NC * NPD,), jnp.float32),
    mesh=_MESH,
    scratch_types=[
        pltpu.VMEM((SBD, CH), jnp.int32),
        pltpu.VMEM((CH,), jnp.float32),
        pltpu.VMEM((DPT,), jnp.float32),
        pltpu.VMEM_SHARED((NPD,), jnp.float32),
        pltpu.SemaphoreType.DMA,
        pltpu.SemaphoreType.DMA,
        pltpu.SemaphoreType.DMA,
        pltpu.SemaphoreType.DMA,
    ],
    compiler_params=pltpu.CompilerParams(use_tc_tiling_on_sc=False),
)


def _sc_msg_body(src_hbm, dst_hbm, tab_hbm, out_hbm, srcb, dstb,
                 r0, r1, r2, r3, zb, acc, g0, g1, g2, g3):
    rows = (r0, r1, r2, r3)
    sg = (g0, g1, g2, g3)
    cid = lax.axis_index("c")
    sid = lax.axis_index("s")
    zeros16 = jnp.zeros((16,), jnp.float32)

    def zfill(i, _):
        zb[i, :] = zeros16
        return 0

    lax.fori_loop(0, ZR, zfill, 0)
    rbase = sid * RPT

    def zcopy(t, _):
        pltpu.sync_copy(zb, acc.at[pl.ds(rbase + t * ZR, ZR)])
        return 0

    lax.fori_loop(0, NZC, zcopy, 0)
    plsc.subcore_barrier()

    crow0 = (cid * NS + sid) * NCHUNK
    for h in range(NSTM):
        pltpu.sync_copy(src_hbm.at[pl.ds(crow0 + h * SBM, SBM)], srcb)
        pltpu.sync_copy(dst_hbm.at[pl.ds(crow0 + h * SBM, SBM)], dstb)
        for b in range(NBUF):
            pltpu.async_copy(tab_hbm.at[srcb.at[b]], rows[b], sg[b])

        def grp(t, _):
            for b in range(NBUF):
                u = t * NBUF + b
                pltpu.make_async_copy(tab_hbm.at[srcb.at[u]], rows[b],
                                      sg[b]).wait()
                pltpu.sync_copy(rows[b], acc.at[dstb.at[u]], add=True)
                pltpu.async_copy(tab_hbm.at[srcb.at[u + NBUF]], rows[b],
                                 sg[b])
            return 0

        lax.fori_loop(0, NGM - 1, grp, 0)
        for b in range(NBUF):
            u = (NGM - 1) * NBUF + b
            pltpu.make_async_copy(tab_hbm.at[srcb.at[u]], rows[b],
                                  sg[b]).wait()
            pltpu.sync_copy(rows[b], acc.at[dstb.at[u]], add=True)
    plsc.subcore_barrier()
    pltpu.sync_copy(acc.at[pl.ds(rbase, RPT)],
                    out_hbm.at[pl.ds(cid * NPAD + rbase, RPT)])


_sc_msg = pl.kernel(
    _sc_msg_body,
    out_type=jax.ShapeDtypeStruct((NC * NPAD, F), jnp.float32),
    mesh=_MESH,
    scratch_types=[
        pltpu.VMEM((SBM, CH), jnp.int32),
        pltpu.VMEM((SBM, CH), jnp.int32),
        pltpu.VMEM((CH, F), jnp.float32),
        pltpu.VMEM((CH, F), jnp.float32),
        pltpu.VMEM((CH, F), jnp.float32),
        pltpu.VMEM((CH, F), jnp.float32),
        pltpu.VMEM((ZR, F), jnp.float32),
        pltpu.VMEM_SHARED((NPAD, F), jnp.float32),
        pltpu.SemaphoreType.DMA,
        pltpu.SemaphoreType.DMA,
        pltpu.SemaphoreType.DMA,
        pltpu.SemaphoreType.DMA,
    ],
    compiler_params=pltpu.CompilerParams(use_tc_tiling_on_sc=False),
)


R = 2048
G = NPAD // R


def _tc_dis_body(deg0_ref, deg1_ref, dis_ref):
    deg = deg0_ref[...] + deg1_ref[...] + 1.0
    dis_ref[...] = lax.rsqrt(deg)


_tc_dis = pl.pallas_call(
    _tc_dis_body,
    grid=(1,),
    in_specs=[
        pl.BlockSpec((NR, 128), lambda i: (0, 0)),
        pl.BlockSpec((NR, 128), lambda i: (1, 0)),
    ],
    out_specs=pl.BlockSpec((NR, 128), lambda i: (0, 0)),
    out_shape=jax.ShapeDtypeStruct((NR, 128), jnp.float32),
)


def _tc_a_body(stc_ref, w1_ref, dis_ref, h1d_ref):
    h1 = jnp.dot(stc_ref[...], w1_ref[...],
                 preferred_element_type=jnp.float32)
    h1d_ref[...] = h1 * dis_ref[...]


_tc_a = pl.pallas_call(
    _tc_a_body,
    grid=(G,),
    in_specs=[
        pl.BlockSpec((R, 18), lambda i: (i, 0)),
        pl.BlockSpec((18, F), lambda i: (0, 0)),
        pl.BlockSpec((R, F), lambda i: (i, 0)),
    ],
    out_specs=pl.BlockSpec((R, F), lambda i: (i, 0)),
    out_shape=jax.ShapeDtypeStruct((N, F), jnp.float32),
)


def _tc_b_body(a0_ref, a1_ref, dis_ref, h1d_ref, b1_ref, w2p_ref, h2d_ref):
    dis = dis_ref[...]
    out1 = dis * (a0_ref[...] + a1_ref[...] + h1d_ref[...]) + b1_ref[...]
    h2 = jnp.dot(out1, w2p_ref[...],
                 preferred_element_type=jnp.float32)
    h2d_ref[...] = h2 * dis


_tc_b = pl.pallas_call(
    _tc_b_body,
    grid=(G,),
    in_specs=[
        pl.BlockSpec((R, F), lambda i: (i, 0)),
        pl.BlockSpec((R, F), lambda i: (G + i, 0)),
        pl.BlockSpec((R, F), lambda i: (i, 0)),
        pl.BlockSpec((R, F), lambda i: (i, 0)),
        pl.BlockSpec((1, F), lambda i: (0, 0)),
        pl.BlockSpec((F, F), lambda i: (0, 0)),
    ],
    out_specs=pl.BlockSpec((R, F), lambda i: (i, 0)),
    out_shape=jax.ShapeDtypeStruct((N, F), jnp.float32),
)


def _tc_c_body(a0_ref, a1_ref, dis_ref, h2d_ref, b2p_ref,
               emba_ref, wca_ref, wcb_ref, bc_ref, out_ref):
    out2 = (dis_ref[...] * (a0_ref[...] + a1_ref[...] + h2d_ref[...])
            + b2p_ref[...])
    out2 = jnp.maximum(out2, 0.0)
    out_ref[...] = (
        jnp.dot(emba_ref[...], wca_ref[...],
                preferred_element_type=jnp.float32)
        + jnp.dot(out2, wcb_ref[...], preferred_element_type=jnp.float32)
        + bc_ref[...])


_tc_c = pl.pallas_call(
    _tc_c_body,
    grid=(G,),
    in_specs=[
        pl.BlockSpec((R, F), lambda i: (i, 0)),
        pl.BlockSpec((R, F), lambda i: (G + i, 0)),
        pl.BlockSpec((R, F), lambda i: (i, 0)),
        pl.BlockSpec((R, F), lambda i: (i, 0)),
        pl.BlockSpec((1, F), lambda i: (0, 0)),
        pl.BlockSpec((R, 40), lambda i: (i, 0)),
        pl.BlockSpec((40, 40), lambda i: (0, 0)),
        pl.BlockSpec((F, 40), lambda i: (0, 0)),
        pl.BlockSpec((1, 40), lambda i: (0, 0)),
    ],
    out_specs=pl.BlockSpec((R, 40), lambda i: (i, 0)),
    out_shape=jax.ShapeDtypeStruct((N, 40), jnp.float32),
)


def kernel(x, edge_index, stc_enc, emb_a, W1, b1, W2, b2, Wc, bc):
    del x
    ei = edge_index.astype(jnp.int32)
    src = jnp.concatenate(
        [ei[0], jnp.zeros((EPAD - E,), jnp.int32)]).reshape(CROWS, CH)
    dst = jnp.concatenate(
        [ei[1], jnp.full((EPAD - E,), N, jnp.int32)]).reshape(CROWS, CH)

    degp = _sc_deg(dst)
    degp2d = degp.reshape(2 * NR, 128)
    dis2d = _tc_dis(degp2d, degp2d)
    dis16 = jnp.broadcast_to(dis2d.reshape(NPD, 1), (NPD, F))

    h1d = _tc_a(stc_enc, W1, dis16)

    acc1 = _sc_msg(src, dst, h1d)
    b1r = b1.reshape(1, F)
    w2p = jnp.concatenate(
        [W2, jnp.zeros((F, F - W2.shape[1]), W2.dtype)], axis=1)
    h2d = _tc_b(acc1, acc1, dis16, h1d, b1r, w2p)

    acc2 = _sc_msg(src, dst, h2d)
    b2p = jnp.concatenate(
        [b2, jnp.zeros((F - b2.shape[0],), b2.dtype)]).reshape(1, F)
    wca = Wc[:40]
    wcb = jnp.concatenate(
        [Wc[40:], jnp.zeros((F - (Wc.shape[0] - 40), 40), Wc.dtype)], axis=0)
    bcr = bc.reshape(1, 40)
    return _tc_c(acc2, acc2, dis16, h2d, b2p, emb_a, wca, wcb, bcr)


# speedup vs baseline: 1.7488x; 1.7488x over previous
import jax
import jax.numpy as jnp
from jax import lax
from jax.experimental import pallas as pl
from jax.experimental.pallas import tpu as pltpu
from jax.experimental.pallas import tpu_sc as plsc

N = 100_000
E = 3_200_000
F = 16
NC = 2
NS = 16
NW = NC * NS
CH = 128
CROWS = 25_600
EPAD = CROWS * CH
NCHUNK = CROWS // NW
NBUF = 4
SBD = 100
NSTD = NCHUNK // SBD
NGD = SBD // NBUF
SBM = 40
NSTM = NCHUNK // SBM
NGM = SBM // NBUF

RPT = 6_272
NPAD = NS * RPT
ZR = 128
NZC = RPT // ZR

DPT = 6_272
NPD = NS * DPT
NR = NPD // 128

_MESH = plsc.VectorSubcoreMesh(core_axis_name="c", subcore_axis_name="s")


def _sc_deg_body(dst_hbm, out_hbm, dstb, onesb, zb, acc, s0, s1, s2, s3):
    ssc = (s0, s1, s2, s3)
    cid = lax.axis_index("c")
    sid = lax.axis_index("s")
    zeros16 = jnp.zeros((16,), jnp.float32)
    ones16 = jnp.ones((16,), jnp.float32)
    for i in range(CH // 16):
        onesb[pl.ds(i * 16, 16)] = ones16

    def zfill(i, _):
        zb[pl.ds(i * 16, 16)] = zeros16
        return 0

    lax.fori_loop(0, DPT // 16, zfill, 0)
    pltpu.sync_copy(zb, acc.at[pl.ds(sid * DPT, DPT)])
    plsc.subcore_barrier()

    crow0 = (cid * NS + sid) * NCHUNK
    for h in range(NSTD):
        pltpu.sync_copy(dst_hbm.at[pl.ds(crow0 + h * SBD, SBD)], dstb)
        for b in range(NBUF):
            pltpu.async_copy(onesb, acc.at[dstb.at[b]], ssc[b], add=True)

        def grp(t, _):
            for b in range(NBUF):
                u = t * NBUF + b
                pltpu.make_async_copy(onesb, acc.at[dstb.at[u]],
                                      ssc[b]).wait()
                pltpu.async_copy(onesb, acc.at[dstb.at[u + NBUF]],
                                 ssc[b], add=True)
            return 0

        lax.fori_loop(0, NGD - 1, grp, 0)
        for b in range(NBUF):
            u = (NGD - 1) * NBUF + b
            pltpu.make_async_copy(onesb, acc.at[dstb.at[u]], ssc[b]).wait()
    plsc.subcore_barrier()
    pltpu.sync_copy(acc.at[pl.ds(sid * DPT, DPT)],
                    out_hbm.at[pl.ds(cid * NPD + sid * DPT, DPT)])


_sc_deg = pl.kernel(
    _sc_deg_body,
    out_type=jax.ShapeDtypeStruct((NC * NPD,), jnp.float32),
    mesh=_MESH,
    scratch_types=[
        pltpu.VMEM((SBD, CH), jnp.int32),
        pltpu.VMEM((CH,), jnp.float32),
        pltpu.VMEM((DPT,), jnp.float32),
        pltpu.VMEM_SHARED((NPD,), jnp.float32),
        pltpu.SemaphoreType.DMA,
        pltpu.SemaphoreType.DMA,
        pltpu.SemaphoreType.DMA,
        pltpu.SemaphoreType.DMA,
    ],
    compiler_params=pltpu.CompilerParams(use_tc_tiling_on_sc=False),
)


def _sc_msg_body(src_hbm, dst_hbm, tab_hbm, out_hbm, srcb, dstb,
                 r0, r1, r2, r3, zb, acc, g0, g1, g2, g3):
    rows = (r0, r1, r2, r3)
    sg = (g0, g1, g2, g3)
    cid = lax.axis_index("c")
    sid = lax.axis_index("s")
    zeros16 = jnp.zeros((16,), jnp.float32)

    def zfill(i, _):
        zb[i, :] = zeros16
        return 0

    lax.fori_loop(0, ZR, zfill, 0)
    rbase = sid * RPT

    def zcopy(t, _):
        pltpu.sync_copy(zb, acc.at[pl.ds(rbase + t * ZR, ZR)])
        return 0

    lax.fori_loop(0, NZC, zcopy, 0)
    plsc.subcore_barrier()

    crow0 = (cid * NS + sid) * NCHUNK
    for h in range(NSTM):
        pltpu.sync_copy(src_hbm.at[pl.ds(crow0 + h * SBM, SBM)], srcb)
        pltpu.sync_copy(dst_hbm.at[pl.ds(crow0 + h * SBM, SBM)], dstb)
        for b in range(NBUF):
            pltpu.async_copy(tab_hbm.at[srcb.at[b]], rows[b], sg[b])

        def grp(t, _):
            for b in range(NBUF):
                u = t * NBUF + b
                pltpu.make_async_copy(tab_hbm.at[srcb.at[u]], rows[b],
                                      sg[b]).wait()
                pltpu.sync_copy(rows[b], acc.at[dstb.at[u]], add=True)
                pltpu.async_copy(tab_hbm.at[srcb.at[u + NBUF]], rows[b],
                                 sg[b])
            return 0

        lax.fori_loop(0, NGM - 1, grp, 0)
        for b in range(NBUF):
            u = (NGM - 1) * NBUF + b
            pltpu.make_async_copy(tab_hbm.at[srcb.at[u]], rows[b],
                                  sg[b]).wait()
            pltpu.sync_copy(rows[b], acc.at[dstb.at[u]], add=True)
    plsc.subcore_barrier()
    pltpu.sync_copy(acc.at[pl.ds(rbase, RPT)],
                    out_hbm.at[pl.ds(cid * NPAD + rbase, RPT)])


_sc_msg = pl.kernel(
    _sc_msg_body,
    out_type=jax.ShapeDtypeStruct((NC * NPAD, F), jnp.float32),
    mesh=_MESH,
    scratch_types=[
        pltpu.VMEM((SBM, CH), jnp.int32),
        pltpu.VMEM((SBM, CH), jnp.int32),
        pltpu.VMEM((CH, F), jnp.float32),
        pltpu.VMEM((CH, F), jnp.float32),
        pltpu.VMEM((CH, F), jnp.float32),
        pltpu.VMEM((CH, F), jnp.float32),
        pltpu.VMEM((ZR, F), jnp.float32),
        pltpu.VMEM_SHARED((NPAD, F), jnp.float32),
        pltpu.SemaphoreType.DMA,
        pltpu.SemaphoreType.DMA,
        pltpu.SemaphoreType.DMA,
        pltpu.SemaphoreType.DMA,
    ],
    compiler_params=pltpu.CompilerParams(use_tc_tiling_on_sc=False),
)


R = 2048
G = NPAD // R


def _tc_dis_body(deg0_ref, deg1_ref, dis_ref):
    deg = deg0_ref[...] + deg1_ref[...] + 1.0
    dis_ref[...] = lax.rsqrt(deg)


_tc_dis = pl.pallas_call(
    _tc_dis_body,
    grid=(1,),
    in_specs=[
        pl.BlockSpec((NR, 128), lambda i: (0, 0)),
        pl.BlockSpec((NR, 128), lambda i: (1, 0)),
    ],
    out_specs=pl.BlockSpec((NR, 128), lambda i: (0, 0)),
    out_shape=jax.ShapeDtypeStruct((NR, 128), jnp.float32),
)


def _tc_a_body(stc_ref, w1_ref, dis_ref, h1d_ref):
    h1 = jnp.dot(stc_ref[...], w1_ref[...],
                 preferred_element_type=jnp.float32)
    h1d_ref[...] = h1 * dis_ref[...]


_tc_a = pl.pallas_call(
    _tc_a_body,
    grid=(G,),
    in_specs=[
        pl.BlockSpec((R, 18), lambda i: (i, 0)),
        pl.BlockSpec((18, F), lambda i: (0, 0)),
        pl.BlockSpec((R, F), lambda i: (i, 0)),
    ],
    out_specs=pl.BlockSpec((R, F), lambda i: (i, 0)),
    out_shape=jax.ShapeDtypeStruct((N, F), jnp.float32),
)


def _tc_b_body(a0_ref, a1_ref, dis_ref, h1d_ref, b1_ref, w2p_ref, h2d_ref):
    dis = dis_ref[...]
    out1 = dis * (a0_ref[...] + a1_ref[...] + h1d_ref[...]) + b1_ref[...]
    h2 = jnp.dot(out1, w2p_ref[...],
                 preferred_element_type=jnp.float32)
    h2d_ref[...] = h2 * dis


_tc_b = pl.pallas_call(
    _tc_b_body,
    grid=(G,),
    in_specs=[
        pl.BlockSpec((R, F), lambda i: (i, 0)),
        pl.BlockSpec((R, F), lambda i: (G + i, 0)),
        pl.BlockSpec((R, F), lambda i: (i, 0)),
        pl.BlockSpec((R, F), lambda i: (i, 0)),
        pl.BlockSpec((1, F), lambda i: (0, 0)),
        pl.BlockSpec((F, F), lambda i: (0, 0)),
    ],
    out_specs=pl.BlockSpec((R, F), lambda i: (i, 0)),
    out_shape=jax.ShapeDtypeStruct((N, F), jnp.float32),
)


def _tc_c_body(a0_ref, a1_ref, dis_ref, h2d_ref, b2p_ref,
               emba_ref, wca_ref, wcb_ref, bc_ref, out_ref):
    out2 = (dis_ref[...] * (a0_ref[...] + a1_ref[...] + h2d_ref[...])
            + b2p_ref[...])
    out2 = jnp.maximum(out2, 0.0)
    out_ref[...] = (
        jnp.dot(emba_ref[...], wca_ref[...],
                preferred_element_type=jnp.float32)
        + jnp.dot(out2, wcb_ref[...], preferred_element_type=jnp.float32)
        + bc_ref[...])


_tc_c = pl.pallas_call(
    _tc_c_body,
    grid=(G,),
    in_specs=[
        pl.BlockSpec((R, F), lambda i: (i, 0)),
        pl.BlockSpec((R, F), lambda i: (G + i, 0)),
        pl.BlockSpec((R, F), lambda i: (i, 0)),
        pl.BlockSpec((R, F), lambda i: (i, 0)),
        pl.BlockSpec((1, F), lambda i: (0, 0)),
        pl.BlockSpec((R, 40), lambda i: (i, 0)),
        pl.BlockSpec((40, 40), lambda i: (0, 0)),
        pl.BlockSpec((F, 40), lambda i: (0, 0)),
        pl.BlockSpec((1, 40), lambda i: (0, 0)),
    ],
    out_specs=pl.BlockSpec((R, 40), lambda i: (i, 0)),
    out_shape=jax.ShapeDtypeStruct((N, 40), jnp.float32),
)


def kernel(x, edge_index, stc_enc, emb_a, W1, b1, W2, b2, Wc, bc):
    del x
    ei = edge_index.astype(jnp.int32)
    padi = jnp.arange(EPAD - E, dtype=jnp.int32) % (NPAD - N)
    src = jnp.concatenate([ei[0], padi]).reshape(CROWS, CH)
    dst = jnp.concatenate([ei[1], N + padi]).reshape(CROWS, CH)

    degp = _sc_deg(dst)
    degp2d = degp.reshape(2 * NR, 128)
    dis2d = _tc_dis(degp2d, degp2d)
    dis16 = jnp.broadcast_to(dis2d.reshape(NPD, 1), (NPD, F))

    h1d = _tc_a(stc_enc, W1, dis16)

    acc1 = _sc_msg(src, dst, h1d)
    b1r = b1.reshape(1, F)
    w2p = jnp.concatenate(
        [W2, jnp.zeros((F, F - W2.shape[1]), W2.dtype)], axis=1)
    h2d = _tc_b(acc1, acc1, dis16, h1d, b1r, w2p)

    acc2 = _sc_msg(src, dst, h2d)
    b2p = jnp.concatenate(
        [b2, jnp.zeros((F - b2.shape[0],), b2.dtype)]).reshape(1, F)
    wca = Wc[:40]
    wcb = jnp.concatenate(
        [Wc[40:], jnp.zeros((F - (Wc.shape[0] - 40), 40), Wc.dtype)], axis=0)
    bcr = bc.reshape(1, 40)
    return _tc_c(acc2, acc2, dis16, h2d, b2p, emb_a, wca, wcb, bcr)


# speedup vs baseline: 1.9722x; 1.1277x over previous
import jax
import jax.numpy as jnp
from jax import lax
from jax.experimental import pallas as pl
from jax.experimental.pallas import tpu as pltpu
from jax.experimental.pallas import tpu_sc as plsc

N = 100_000
E = 3_200_000
F = 16
NC = 2
NS = 16
NW = NC * NS
CH = 128
CROWS = 25_600
EPAD = CROWS * CH
NCHUNK = CROWS // NW
NBUF = 4
MBUF = 8
SBD = 100
NSTD = NCHUNK // SBD
NGD = SBD // NBUF
SBM = 40
NSTM = NCHUNK // SBM
NGM = SBM // MBUF

RPT = 6_272
NPAD = NS * RPT
ZR = 128
NZC = RPT // ZR

DPT = 6_272
NPD = NS * DPT
NR = NPD // 128

_MESH = plsc.VectorSubcoreMesh(core_axis_name="c", subcore_axis_name="s")


def _sc_deg_body(dst_hbm, out_hbm, dstb, onesb, zb, acc, s0, s1, s2, s3):
    ssc = (s0, s1, s2, s3)
    cid = lax.axis_index("c")
    sid = lax.axis_index("s")
    zeros16 = jnp.zeros((16,), jnp.float32)
    ones16 = jnp.ones((16,), jnp.float32)
    for i in range(CH // 16):
        onesb[pl.ds(i * 16, 16)] = ones16

    def zfill(i, _):
        zb[pl.ds(i * 16, 16)] = zeros16
        return 0

    lax.fori_loop(0, DPT // 16, zfill, 0)
    pltpu.sync_copy(zb, acc.at[pl.ds(sid * DPT, DPT)])
    plsc.subcore_barrier()

    crow0 = (cid * NS + sid) * NCHUNK
    for h in range(NSTD):
        pltpu.sync_copy(dst_hbm.at[pl.ds(crow0 + h * SBD, SBD)], dstb)
        for b in range(NBUF):
            pltpu.async_copy(onesb, acc.at[dstb.at[b]], ssc[b], add=True)

        def grp(t, _):
            for b in range(NBUF):
                u = t * NBUF + b
                pltpu.make_async_copy(onesb, acc.at[dstb.at[u]],
                                      ssc[b]).wait()
                pltpu.async_copy(onesb, acc.at[dstb.at[u + NBUF]],
                                 ssc[b], add=True)
            return 0

        lax.fori_loop(0, NGD - 1, grp, 0)
        for b in range(NBUF):
            u = (NGD - 1) * NBUF + b
            pltpu.make_async_copy(onesb, acc.at[dstb.at[u]], ssc[b]).wait()
    plsc.subcore_barrier()
    pltpu.sync_copy(acc.at[pl.ds(sid * DPT, DPT)],
                    out_hbm.at[pl.ds(cid * NPD + sid * DPT, DPT)])


_sc_deg = pl.kernel(
    _sc_deg_body,
    out_type=jax.ShapeDtypeStruct((NC * NPD,), jnp.float32),
    mesh=_MESH,
    scratch_types=[
        pltpu.VMEM((SBD, CH), jnp.int32),
        pltpu.VMEM((CH,), jnp.float32),
        pltpu.VMEM((DPT,), jnp.float32),
        pltpu.VMEM_SHARED((NPD,), jnp.float32),
        pltpu.SemaphoreType.DMA,
        pltpu.SemaphoreType.DMA,
        pltpu.SemaphoreType.DMA,
        pltpu.SemaphoreType.DMA,
    ],
    compiler_params=pltpu.CompilerParams(use_tc_tiling_on_sc=False),
)


def _sc_msg_body(src_hbm, dst_hbm, tab_hbm, out_hbm, srcb, dstb,
                 r0, r1, r2, r3, r4, r5, r6, r7, zb, acc,
                 g0, g1, g2, g3, g4, g5, g6, g7):
    rows = (r0, r1, r2, r3, r4, r5, r6, r7)
    sg = (g0, g1, g2, g3, g4, g5, g6, g7)
    cid = lax.axis_index("c")
    sid = lax.axis_index("s")
    zeros16 = jnp.zeros((16,), jnp.float32)

    def zfill(i, _):
        zb[i, :] = zeros16
        return 0

    lax.fori_loop(0, ZR, zfill, 0)
    rbase = sid * RPT

    def zcopy(t, _):
        pltpu.sync_copy(zb, acc.at[pl.ds(rbase + t * ZR, ZR)])
        return 0

    lax.fori_loop(0, NZC, zcopy, 0)
    plsc.subcore_barrier()

    crow0 = (cid * NS + sid) * NCHUNK
    for h in range(NSTM):
        pltpu.sync_copy(src_hbm.at[pl.ds(crow0 + h * SBM, SBM)], srcb)
        pltpu.sync_copy(dst_hbm.at[pl.ds(crow0 + h * SBM, SBM)], dstb)
        for b in range(MBUF):
            pltpu.async_copy(tab_hbm.at[srcb.at[b]], rows[b], sg[b])

        def grp(t, _):
            for b in range(MBUF):
                u = t * MBUF + b
                pltpu.make_async_copy(tab_hbm.at[srcb.at[u]], rows[b],
                                      sg[b]).wait()
                pltpu.sync_copy(rows[b], acc.at[dstb.at[u]], add=True)
                pltpu.async_copy(tab_hbm.at[srcb.at[u + MBUF]], rows[b],
                                 sg[b])
            return 0

        lax.fori_loop(0, NGM - 1, grp, 0)
        for b in range(MBUF):
            u = (NGM - 1) * MBUF + b
            pltpu.make_async_copy(tab_hbm.at[srcb.at[u]], rows[b],
                                  sg[b]).wait()
            pltpu.sync_copy(rows[b], acc.at[dstb.at[u]], add=True)
    plsc.subcore_barrier()
    pltpu.sync_copy(acc.at[pl.ds(rbase, RPT)],
                    out_hbm.at[pl.ds(cid * NPAD + rbase, RPT)])


_sc_msg = pl.kernel(
    _sc_msg_body,
    out_type=jax.ShapeDtypeStruct((NC * NPAD, F), jnp.float32),
    mesh=_MESH,
    scratch_types=[
        pltpu.VMEM((SBM, CH), jnp.int32),
        pltpu.VMEM((SBM, CH), jnp.int32),
        pltpu.VMEM((CH, F), jnp.float32),
        pltpu.VMEM((CH, F), jnp.float32),
        pltpu.VMEM((CH, F), jnp.float32),
        pltpu.VMEM((CH, F), jnp.float32),
        pltpu.VMEM((CH, F), jnp.float32),
        pltpu.VMEM((CH, F), jnp.float32),
        pltpu.VMEM((CH, F), jnp.float32),
        pltpu.VMEM((CH, F), jnp.float32),
        pltpu.VMEM((ZR, F), jnp.float32),
        pltpu.VMEM_SHARED((NPAD, F), jnp.float32),
        pltpu.SemaphoreType.DMA,
        pltpu.SemaphoreType.DMA,
        pltpu.SemaphoreType.DMA,
        pltpu.SemaphoreType.DMA,
        pltpu.SemaphoreType.DMA,
        pltpu.SemaphoreType.DMA,
        pltpu.SemaphoreType.DMA,
        pltpu.SemaphoreType.DMA,
    ],
    compiler_params=pltpu.CompilerParams(use_tc_tiling_on_sc=False),
)


R = 2048
G = NPAD // R


def _tc_dis_body(deg0_ref, deg1_ref, dis_ref):
    deg = deg0_ref[...] + deg1_ref[...] + 1.0
    dis_ref[...] = lax.rsqrt(deg)


_tc_dis = pl.pallas_call(
    _tc_dis_body,
    grid=(1,),
    in_specs=[
        pl.BlockSpec((NR, 128), lambda i: (0, 0)),
        pl.BlockSpec((NR, 128), lambda i: (1, 0)),
    ],
    out_specs=pl.BlockSpec((NR, 128), lambda i: (0, 0)),
    out_shape=jax.ShapeDtypeStruct((NR, 128), jnp.float32),
)


def _tc_a_body(stc_ref, w1_ref, dis_ref, h1d_ref):
    h1 = jnp.dot(stc_ref[...], w1_ref[...],
                 preferred_element_type=jnp.float32)
    h1d_ref[...] = h1 * dis_ref[...]


_tc_a = pl.pallas_call(
    _tc_a_body,
    grid=(G,),
    in_specs=[
        pl.BlockSpec((R, 18), lambda i: (i, 0)),
        pl.BlockSpec((18, F), lambda i: (0, 0)),
        pl.BlockSpec((R, F), lambda i: (i, 0)),
    ],
    out_specs=pl.BlockSpec((R, F), lambda i: (i, 0)),
    out_shape=jax.ShapeDtypeStruct((N, F), jnp.float32),
)


def _tc_b_body(a0_ref, a1_ref, dis_ref, h1d_ref, b1_ref, w2p_ref, h2d_ref):
    dis = dis_ref[...]
    out1 = dis * (a0_ref[...] + a1_ref[...] + h1d_ref[...]) + b1_ref[...]
    h2 = jnp.dot(out1, w2p_ref[...],
                 preferred_element_type=jnp.float32)
    h2d_ref[...] = h2 * dis


_tc_b = pl.pallas_call(
    _tc_b_body,
    grid=(G,),
    in_specs=[
        pl.BlockSpec((R, F), lambda i: (i, 0)),
        pl.BlockSpec((R, F), lambda i: (G + i, 0)),
        pl.BlockSpec((R, F), lambda i: (i, 0)),
        pl.BlockSpec((R, F), lambda i: (i, 0)),
        pl.BlockSpec((1, F), lambda i: (0, 0)),
        pl.BlockSpec((F, F), lambda i: (0, 0)),
    ],
    out_specs=pl.BlockSpec((R, F), lambda i: (i, 0)),
    out_shape=jax.ShapeDtypeStruct((N, F), jnp.float32),
)


def _tc_c_body(a0_ref, a1_ref, dis_ref, h2d_ref, b2p_ref,
               emba_ref, wca_ref, wcb_ref, bc_ref, out_ref):
    out2 = (dis_ref[...] * (a0_ref[...] + a1_ref[...] + h2d_ref[...])
            + b2p_ref[...])
    out2 = jnp.maximum(out2, 0.0)
    out_ref[...] = (
        jnp.dot(emba_ref[...], wca_ref[...],
                preferred_element_type=jnp.float32)
        + jnp.dot(out2, wcb_ref[...], preferred_element_type=jnp.float32)
        + bc_ref[...])


_tc_c = pl.pallas_call(
    _tc_c_body,
    grid=(G,),
    in_specs=[
        pl.BlockSpec((R, F), lambda i: (i, 0)),
        pl.BlockSpec((R, F), lambda i: (G + i, 0)),
        pl.BlockSpec((R, F), lambda i: (i, 0)),
        pl.BlockSpec((R, F), lambda i: (i, 0)),
        pl.BlockSpec((1, F), lambda i: (0, 0)),
        pl.BlockSpec((R, 40), lambda i: (i, 0)),
        pl.BlockSpec((40, 40), lambda i: (0, 0)),
        pl.BlockSpec((F, 40), lambda i: (0, 0)),
        pl.BlockSpec((1, 40), lambda i: (0, 0)),
    ],
    out_specs=pl.BlockSpec((R, 40), lambda i: (i, 0)),
    out_shape=jax.ShapeDtypeStruct((N, 40), jnp.float32),
)


def kernel(x, edge_index, stc_enc, emb_a, W1, b1, W2, b2, Wc, bc):
    del x
    ei = edge_index.astype(jnp.int32)
    padi = jnp.arange(EPAD - E, dtype=jnp.int32) % (NPAD - N)
    src = jnp.concatenate([ei[0], padi]).reshape(CROWS, CH)
    dst = jnp.concatenate([ei[1], N + padi]).reshape(CROWS, CH)

    degp = _sc_deg(dst)
    degp2d = degp.reshape(2 * NR, 128)
    dis2d = _tc_dis(degp2d, degp2d)
    dis16 = jnp.broadcast_to(dis2d.reshape(NPD, 1), (NPD, F))

    h1d = _tc_a(stc_enc, W1, dis16)

    acc1 = _sc_msg(src, dst, h1d)
    b1r = b1.reshape(1, F)
    w2p = jnp.concatenate(
        [W2, jnp.zeros((F, F - W2.shape[1]), W2.dtype)], axis=1)
    h2d = _tc_b(acc1, acc1, dis16, h1d, b1r, w2p)

    acc2 = _sc_msg(src, dst, h2d)
    b2p = jnp.concatenate(
        [b2, jnp.zeros((F - b2.shape[0],), b2.dtype)]).reshape(1, F)
    wca = Wc[:40]
    wcb = jnp.concatenate(
        [Wc[40:], jnp.zeros((F - (Wc.shape[0] - 40), 40), Wc.dtype)], axis=0)
    bcr = bc.reshape(1, 40)
    return _tc_c(acc2, acc2, dis16, h2d, b2p, emb_a, wca, wcb, bcr)
